# baseline (speedup 1.0000x reference)
import numpy as np
import jax
import jax.numpy as jnp
from jax import lax
from jax.experimental import pallas as pl

_ANCHORS = np.array([[12.0, 16.0], [19.0, 36.0], [40.0, 28.0]], dtype=np.float32)
_A = 3
_C = 85
_G = 76
_HW = _G * _G


def _body(s_ref, x_ref, o_ref):
    stride = s_ref[0, 0]
    col = lax.broadcasted_iota(jnp.int32, (1, _HW), 1)
    gx = (col % _G).astype(jnp.float32)
    gy = (col // _G).astype(jnp.float32)
    v = x_ref[0].reshape(_A * _C, _HW)
    pieces = []
    zpad = jnp.zeros((128 - _C, _HW), jnp.float32)
    for a in range(_A):
        va = v[a * _C:(a + 1) * _C, :]
        sg = 0.5 * jnp.tanh(va * 0.5) + 0.5
        r0 = (sg[0:1] + gx) * stride
        r1 = (sg[1:2] + gy) * stride
        r2 = jnp.exp(va[2:3]) * float(_ANCHORS[a, 0])
        r3 = jnp.exp(va[3:4]) * float(_ANCHORS[a, 1])
        pieces += [r0, r1, r2, r3, sg[4:], zpad]
    fullp = jnp.concatenate(pieces, axis=0)
    tp = fullp.T
    inter = tp.reshape(_HW, _A, 128).reshape(_HW * _A, 128)
    o_ref[0] = inter[:, :_C]


def kernel(x, input_dim):
    b, c, g, _ = x.shape
    hw = g * g
    stride = jnp.floor_divide(input_dim, g).astype(jnp.float32).reshape(1, 1)
    out = pl.pallas_call(
        _body,
        grid=(b,),
        in_specs=[
            pl.BlockSpec((1, 1), lambda i: (0, 0)),
            pl.BlockSpec((1, c, g, g), lambda i: (i, 0, 0, 0)),
        ],
        out_specs=pl.BlockSpec((1, hw * _A, _C), lambda i: (i, 0, 0)),
        out_shape=jax.ShapeDtypeStruct((b, hw * _A, _C), jnp.float32),
    )(stride, x)
    return out


# speedup vs baseline: 1.1978x; 1.1978x over previous
import numpy as np
import jax
import jax.numpy as jnp
from jax import lax
from jax.experimental import pallas as pl

_ANCHORS = np.array([[12.0, 16.0], [19.0, 36.0], [40.0, 28.0]], dtype=np.float32)
_A = 3
_C = 85
_G = 76
_HW = _G * _G


def _body(s_ref, x_ref, o_ref):
    stride = s_ref[0, 0]
    col = lax.broadcasted_iota(jnp.int32, (1, _HW), 1)
    gx = (col % _G).astype(jnp.float32)
    gy = (col // _G).astype(jnp.float32)
    v = x_ref[0]
    pieces = []
    zpad = jnp.zeros((128 - _C, _HW), jnp.float32)
    for a in range(_A):
        va = v[a * _C:(a + 1) * _C, :]
        sg = 0.5 * jnp.tanh(va * 0.5) + 0.5
        r0 = (sg[0:1] + gx) * stride
        r1 = (sg[1:2] + gy) * stride
        r2 = jnp.exp(va[2:3]) * float(_ANCHORS[a, 0])
        r3 = jnp.exp(va[3:4]) * float(_ANCHORS[a, 1])
        pieces += [r0, r1, r2, r3, sg[4:], zpad]
    fullp = jnp.concatenate(pieces, axis=0)
    tp = fullp.T
    inter = tp.reshape(_HW, _A, 128).reshape(_HW * _A, 128)
    o_ref[0] = inter[:, :_C]


def kernel(x, input_dim):
    b, c, g, _ = x.shape
    hw = g * g
    xr = x.reshape(b, c, hw)
    stride = jnp.floor_divide(input_dim, g).astype(jnp.float32).reshape(1, 1)
    out = pl.pallas_call(
        _body,
        grid=(b,),
        in_specs=[
            pl.BlockSpec((1, 1), lambda i: (0, 0)),
            pl.BlockSpec((1, c, hw), lambda i: (i, 0, 0)),
        ],
        out_specs=pl.BlockSpec((1, hw * _A, _C), lambda i: (i, 0, 0)),
        out_shape=jax.ShapeDtypeStruct((b, hw * _A, _C), jnp.float32),
    )(stride, xr)
    return out
